# Initial kernel scaffold; baseline (speedup 1.0000x reference)
import jax
import jax.numpy as jnp
from jax.experimental import pallas as pl


def kernel(x, edge_index, W_rel_src, W_rel_dst, b_rel, W_msg, W_self, W_out, b_out):
    raise NotImplementedError("write your pallas kernel here")


# speedup vs baseline: 5.9689x; 5.9689x over previous
import functools

import jax
import jax.numpy as jnp
from jax import lax
from jax.experimental import pallas as pl
from jax.experimental.pallas import tpu as pltpu
from jax.experimental.pallas import tpu_sc as plsc

N = 10000
E = 320000
D = 128
O = 64

NC = 2
NS = 16
NW = NC * NS
EPW = E // NW
CH = 16
NCHUNK = EPW // CH
NP = 10240
ROWS_PER_TILE = NP // NS
ZROWS = 32

BN = 2000


def _stage_a_body(x_ref, wmsg_ref, wself_ref, wd_ref, cvec_ref,
                  xm_ref, xs_ref, ab_ref):
    xb = x_ref[...]
    xm_ref[...] = jnp.dot(xb, wmsg_ref[...], preferred_element_type=jnp.float32)
    xs_ref[...] = jnp.dot(xb, wself_ref[...], preferred_element_type=jnp.float32)
    ab_ref[...] = (jnp.dot(xb, wd_ref[...], preferred_element_type=jnp.float32)
                   + cvec_ref[...])


def _stage_a(x, w_msg, w_self, wd, cvec):
    return pl.pallas_call(
        _stage_a_body,
        grid=(N // BN,),
        in_specs=[
            pl.BlockSpec((BN, D), lambda i: (i, 0)),
            pl.BlockSpec((D, D), lambda i: (0, 0)),
            pl.BlockSpec((D, D), lambda i: (0, 0)),
            pl.BlockSpec((D, 2), lambda i: (0, 0)),
            pl.BlockSpec((1, 2), lambda i: (0, 0)),
        ],
        out_specs=[
            pl.BlockSpec((BN, D), lambda i: (i, 0)),
            pl.BlockSpec((BN, D), lambda i: (i, 0)),
            pl.BlockSpec((BN, 2), lambda i: (i, 0)),
        ],
        out_shape=[
            jax.ShapeDtypeStruct((N, D), jnp.float32),
            jax.ShapeDtypeStruct((N, D), jnp.float32),
            jax.ShapeDtypeStruct((N, 2), jnp.float32),
        ],
    )(x, w_msg, w_self, wd, cvec)


def _stage_b_body(xs_ref, agg_ref, wout_ref, bout_ref, out_ref):
    acc = xs_ref[...] + agg_ref[0] + agg_ref[1]
    h = jnp.maximum(acc, 0.0)
    out_ref[...] = (jnp.dot(h, wout_ref[...], preferred_element_type=jnp.float32)
                    + bout_ref[...])


def _stage_b(xs, agg2, w_out, bout2):
    return pl.pallas_call(
        _stage_b_body,
        grid=(N // BN,),
        in_specs=[
            pl.BlockSpec((BN, D), lambda i: (i, 0)),
            pl.BlockSpec((NC, BN, D), lambda i: (0, i, 0)),
            pl.BlockSpec((D, O), lambda i: (0, 0)),
            pl.BlockSpec((1, O), lambda i: (0, 0)),
        ],
        out_specs=pl.BlockSpec((BN, O), lambda i: (i, 0)),
        out_shape=jax.ShapeDtypeStruct((N, O), jnp.float32),
    )(xs, agg2, w_out, bout2)


def _sc_edge_body(xm_hbm, ab_hbm, edge_hbm, out_hbm,
                  src_v, dst_v, ab_v, rows_v, sidx_v, wtmp_v, z_v, agg_sh):
    cid = lax.axis_index("c")
    sid = lax.axis_index("s")
    wid = cid * NS + sid
    ebase = wid * EPW

    pltpu.sync_copy(edge_hbm.at[pl.ds(ebase, EPW)], src_v)
    pltpu.sync_copy(edge_hbm.at[pl.ds(E + ebase, EPW)], dst_v)
    pltpu.sync_copy(ab_hbm, ab_v)

    zeros16 = jnp.zeros((16,), jnp.float32)
    for r in range(ZROWS):
        for c in range(D // 16):
            z_v[r, pl.ds(c * 16, 16)] = zeros16
    rbase = sid * ROWS_PER_TILE

    def zero_loop(i, carry):
        pltpu.sync_copy(z_v, agg_sh.at[pl.ds(rbase + i * ZROWS, ZROWS)])
        return carry

    lax.fori_loop(0, ROWS_PER_TILE // ZROWS, zero_loop, 0)
    plsc.subcore_barrier()

    def chunk(g, carry):
        sv = src_v[pl.ds(g * CH, CH)]
        dv = dst_v[pl.ds(g * CH, CH)]
        av = plsc.load_gather(ab_v, [sv * 2])
        bv = plsc.load_gather(ab_v, [dv * 2 + 1])
        z = av + bv
        p = 1.0 / (1.0 + jnp.exp(-z))
        w = jnp.where(sv <= dv, p, 0.0)
        sidx_v[...] = dv
        pltpu.sync_copy(xm_hbm.at[src_v.at[pl.ds(g * CH, CH)]], rows_v)
        for j in range(CH):
            s = w[j]
            for c in range(D // 16):
                rows_v[j, pl.ds(c * 16, 16)] = rows_v[j, pl.ds(c * 16, 16)] * s
        pltpu.sync_copy(rows_v, agg_sh.at[sidx_v], add=True)
        return carry

    lax.fori_loop(0, NCHUNK, chunk, 0)
    plsc.subcore_barrier()

    pltpu.sync_copy(agg_sh.at[pl.ds(rbase, ROWS_PER_TILE)],
                    out_hbm.at[cid, pl.ds(rbase, ROWS_PER_TILE)])


def _sc_edge(xm, ab, edge_index):
    mesh = plsc.VectorSubcoreMesh(core_axis_name="c", subcore_axis_name="s")
    f = pl.kernel(
        _sc_edge_body,
        out_type=jax.ShapeDtypeStruct((NC, NP, D), jnp.float32),
        mesh=mesh,
        compiler_params=pltpu.CompilerParams(needs_layout_passes=False),
        scratch_types=[
            pltpu.VMEM((EPW,), jnp.int32),
            pltpu.VMEM((EPW,), jnp.int32),
            pltpu.VMEM((2 * N,), jnp.float32),
            pltpu.VMEM((CH, D), jnp.float32),
            pltpu.VMEM((CH,), jnp.int32),
            pltpu.VMEM((CH,), jnp.float32),
            pltpu.VMEM((ZROWS, D), jnp.float32),
            pltpu.VMEM_SHARED((NP, D), jnp.float32),
        ],
    )
    return f(xm, ab, edge_index)


def kernel(x, edge_index, W_rel_src, W_rel_dst, b_rel, W_msg, W_self, W_out,
           b_out):
    da = W_rel_src[:, 1] - W_rel_src[:, 0]
    db = W_rel_dst[:, 1] - W_rel_dst[:, 0]
    wd = jnp.stack([da, db], axis=1)
    c0 = (b_rel[1] - b_rel[0]).reshape(1, 1)
    cvec = jnp.concatenate([c0, jnp.zeros((1, 1), jnp.float32)], axis=1)
    xm, xs, ab = _stage_a(x, W_msg, W_self, wd, cvec)
    agg2 = _sc_edge(xm, ab.reshape(-1), edge_index.reshape(-1))
    return _stage_b(xs, agg2, W_out, b_out.reshape(1, O))


# speedup vs baseline: 15.5514x; 2.6054x over previous
import functools

import jax
import jax.numpy as jnp
from jax import lax
from jax.experimental import pallas as pl
from jax.experimental.pallas import tpu as pltpu
from jax.experimental.pallas import tpu_sc as plsc

N = 10000
E = 320000
D = 128
O = 64

NC = 2
NS = 16
NW = NC * NS
EPW = E // NW
CB = 16
NBUF = 4
NCHB = EPW // CB
NP = 10112
ROWS_PER_TILE = NP // NS

BN = 2000


def _stage_a_body(x_ref, wmsg_ref, wself_ref, wd_ref, cvec_ref,
                  xm_ref, xs_ref, ab_ref):
    xb = x_ref[...]
    xm_ref[...] = jnp.dot(xb, wmsg_ref[...], preferred_element_type=jnp.float32)
    xs_ref[...] = jnp.dot(xb, wself_ref[...], preferred_element_type=jnp.float32)
    ab_ref[...] = (jnp.dot(xb, wd_ref[...], preferred_element_type=jnp.float32)
                   + cvec_ref[...])


def _stage_a(x, w_msg, w_self, wd, cvec):
    return pl.pallas_call(
        _stage_a_body,
        grid=(N // BN,),
        in_specs=[
            pl.BlockSpec((BN, D), lambda i: (i, 0)),
            pl.BlockSpec((D, D), lambda i: (0, 0)),
            pl.BlockSpec((D, D), lambda i: (0, 0)),
            pl.BlockSpec((D, 2), lambda i: (0, 0)),
            pl.BlockSpec((1, 2), lambda i: (0, 0)),
        ],
        out_specs=[
            pl.BlockSpec((BN, D), lambda i: (i, 0)),
            pl.BlockSpec((BN, D), lambda i: (i, 0)),
            pl.BlockSpec((BN, 2), lambda i: (i, 0)),
        ],
        out_shape=[
            jax.ShapeDtypeStruct((N, D), jnp.float32),
            jax.ShapeDtypeStruct((N, D), jnp.float32),
            jax.ShapeDtypeStruct((N, 2), jnp.float32),
        ],
    )(x, w_msg, w_self, wd, cvec)


def _stage_b_body(xs_ref, agg_ref, wout_ref, bout_ref, out_ref):
    acc = xs_ref[...] + agg_ref[0] + agg_ref[1]
    h = jnp.maximum(acc, 0.0)
    out_ref[...] = (jnp.dot(h, wout_ref[...], preferred_element_type=jnp.float32)
                    + bout_ref[...])


def _stage_b(xs, agg2, w_out, bout2):
    return pl.pallas_call(
        _stage_b_body,
        grid=(N // BN,),
        in_specs=[
            pl.BlockSpec((BN, D), lambda i: (i, 0)),
            pl.BlockSpec((NC, BN, D), lambda i: (0, i, 0)),
            pl.BlockSpec((D, O), lambda i: (0, 0)),
            pl.BlockSpec((1, O), lambda i: (0, 0)),
        ],
        out_specs=pl.BlockSpec((BN, O), lambda i: (i, 0)),
        out_shape=jax.ShapeDtypeStruct((N, O), jnp.float32),
    )(xs, agg2, w_out, bout2)


def _sc_edge_body(xm_hbm, ab_hbm, edge_hbm, out_hbm,
                  src_v, dst_v, ab_v, rows_v, sidx_v, agg_sh, gsem, ssem):
    cid = lax.axis_index("c")
    sid = lax.axis_index("s")
    wid = cid * NS + sid
    ebase = wid * EPW

    pltpu.sync_copy(edge_hbm.at[pl.ds(ebase, EPW)], src_v)
    pltpu.sync_copy(edge_hbm.at[pl.ds(E + ebase, EPW)], dst_v)
    pltpu.sync_copy(ab_hbm, ab_v)

    zeros16 = jnp.zeros((16,), jnp.float32)
    for r in range(8):
        for c in range(D // 16):
            rows_v[0, r, pl.ds(c * 16, 16)] = zeros16
    rbase = sid * ROWS_PER_TILE

    def zero_loop(i, carry):
        pltpu.sync_copy(rows_v.at[0, pl.ds(0, 8)],
                        agg_sh.at[pl.ds(rbase + i * 8, 8)])
        return carry

    lax.fori_loop(0, ROWS_PER_TILE // 8, zero_loop, 0)
    plsc.subcore_barrier()

    def gather_start(g, b):
        pltpu.async_copy(xm_hbm.at[src_v.at[pl.ds(g * CB, CB)]],
                         rows_v.at[b], gsem.at[b])

    def gather_wait(g, b):
        pltpu.make_async_copy(xm_hbm.at[src_v.at[pl.ds(g * CB, CB)]],
                              rows_v.at[b], gsem.at[b]).wait()

    def scatter_start(b):
        pltpu.async_copy(rows_v.at[b], agg_sh.at[sidx_v.at[b]], ssem.at[b],
                         add=True)

    def scatter_wait(b):
        pltpu.make_async_copy(rows_v.at[b], agg_sh.at[sidx_v.at[b]],
                              ssem.at[b]).wait()

    for b in range(NBUF):
        gather_start(b, b)

    def slot(g, carry):
        b = lax.rem(g, NBUF)
        gather_wait(g, b)
        sv = src_v[pl.ds(g * CB, CB)]
        dv = dst_v[pl.ds(g * CB, CB)]
        av = plsc.load_gather(ab_v, [sv * 2])
        bv = plsc.load_gather(ab_v, [dv * 2 + 1])
        p = 1.0 / (1.0 + jnp.exp(-(av + bv)))
        w = jnp.where(sv <= dv, p, 0.0)
        sidx_v[b, pl.ds(0, CB)] = dv
        for j in range(CB):
            s = w[j]
            for c in range(D // 16):
                rows_v[b, j, pl.ds(c * 16, 16)] = (
                    rows_v[b, j, pl.ds(c * 16, 16)] * s)
        scatter_start(b)
        bp = lax.rem(b + NBUF - 1, NBUF)
        gn = g - 1 + NBUF

        @pl.when(jnp.logical_and(g >= 1, gn < NCHB))
        def _():
            scatter_wait(bp)
            gather_start(gn, bp)

        return carry

    lax.fori_loop(0, NCHB, slot, 0)
    for b in range(NBUF):
        scatter_wait(b)
    plsc.subcore_barrier()

    pltpu.sync_copy(agg_sh.at[pl.ds(rbase, ROWS_PER_TILE)],
                    out_hbm.at[cid, pl.ds(rbase, ROWS_PER_TILE)])


def _sc_edge(xm, ab, edge_index):
    mesh = plsc.VectorSubcoreMesh(core_axis_name="c", subcore_axis_name="s")
    f = pl.kernel(
        _sc_edge_body,
        out_type=jax.ShapeDtypeStruct((NC, NP, D), jnp.float32),
        mesh=mesh,
        compiler_params=pltpu.CompilerParams(needs_layout_passes=False),
        scratch_types=[
            pltpu.VMEM((EPW,), jnp.int32),
            pltpu.VMEM((EPW,), jnp.int32),
            pltpu.VMEM((2 * N,), jnp.float32),
            pltpu.VMEM((NBUF, CB, D), jnp.float32),
            pltpu.VMEM((NBUF, CB), jnp.int32),
            pltpu.VMEM_SHARED((NP, D), jnp.float32),
            pltpu.SemaphoreType.DMA((NBUF,)),
            pltpu.SemaphoreType.DMA((NBUF,)),
        ],
    )
    return f(xm, ab, edge_index)


def kernel(x, edge_index, W_rel_src, W_rel_dst, b_rel, W_msg, W_self, W_out,
           b_out):
    da = W_rel_src[:, 1] - W_rel_src[:, 0]
    db = W_rel_dst[:, 1] - W_rel_dst[:, 0]
    wd = jnp.stack([da, db], axis=1)
    c0 = (b_rel[1] - b_rel[0]).reshape(1, 1)
    cvec = jnp.concatenate([c0, jnp.zeros((1, 1), jnp.float32)], axis=1)
    xm, xs, ab = _stage_a(x, W_msg, W_self, wd, cvec)
    agg2 = _sc_edge(xm, ab.reshape(-1), edge_index.reshape(-1))
    return _stage_b(xs, agg2, W_out, b_out.reshape(1, O))


# speedup vs baseline: 22.6775x; 1.4582x over previous
import functools

import jax
import jax.numpy as jnp
from jax import lax
from jax.experimental import pallas as pl
from jax.experimental.pallas import tpu as pltpu
from jax.experimental.pallas import tpu_sc as plsc

N = 10000
E = 320000
D = 128
O = 64

NC = 2
NS = 16
NW = NC * NS
EPW = E // NW
CB = 16
NBUF = 4
NCHB = EPW // CB
NP = 10112
ROWS_PER_TILE = NP // NS

BN = 2000


def _stage_a_body(x_ref, wmsg_ref, wself_ref, wd_ref, cvec_ref,
                  xm_ref, xs_ref, ab_ref):
    xb = x_ref[...]
    xm_ref[...] = jnp.dot(xb, wmsg_ref[...], preferred_element_type=jnp.float32)
    xs_ref[...] = jnp.dot(xb, wself_ref[...], preferred_element_type=jnp.float32)
    ab_ref[...] = (jnp.dot(xb, wd_ref[...], preferred_element_type=jnp.float32)
                   + cvec_ref[...])


def _stage_a(x, w_msg, w_self, wd, cvec):
    return pl.pallas_call(
        _stage_a_body,
        grid=(N // BN,),
        in_specs=[
            pl.BlockSpec((BN, D), lambda i: (i, 0)),
            pl.BlockSpec((D, D), lambda i: (0, 0)),
            pl.BlockSpec((D, D), lambda i: (0, 0)),
            pl.BlockSpec((D, 2), lambda i: (0, 0)),
            pl.BlockSpec((1, 2), lambda i: (0, 0)),
        ],
        out_specs=[
            pl.BlockSpec((BN, D), lambda i: (i, 0)),
            pl.BlockSpec((BN, D), lambda i: (i, 0)),
            pl.BlockSpec((BN, 2), lambda i: (i, 0)),
        ],
        out_shape=[
            jax.ShapeDtypeStruct((N, D), jnp.float32),
            jax.ShapeDtypeStruct((N, D), jnp.float32),
            jax.ShapeDtypeStruct((N, 2), jnp.float32),
        ],
    )(x, w_msg, w_self, wd, cvec)


def _stage_b_body(xs_ref, agg_ref, wout_ref, bout_ref, out_ref):
    acc = xs_ref[...] + agg_ref[0] + agg_ref[1]
    h = jnp.maximum(acc, 0.0)
    out_ref[...] = (jnp.dot(h, wout_ref[...], preferred_element_type=jnp.float32)
                    + bout_ref[...])


def _stage_b(xs, agg2, w_out, bout2):
    return pl.pallas_call(
        _stage_b_body,
        grid=(N // BN,),
        in_specs=[
            pl.BlockSpec((BN, D), lambda i: (i, 0)),
            pl.BlockSpec((NC, BN, D), lambda i: (0, i, 0)),
            pl.BlockSpec((D, O), lambda i: (0, 0)),
            pl.BlockSpec((1, O), lambda i: (0, 0)),
        ],
        out_specs=pl.BlockSpec((BN, O), lambda i: (i, 0)),
        out_shape=jax.ShapeDtypeStruct((N, O), jnp.float32),
    )(xs, agg2, w_out, bout2)


def _sc_edge_body(xm_hbm, ab_hbm, edge_hbm, out_hbm,
                  src_v, dst_v, ab_v, rows_v, sidx_v, agg_sh, gsem, ssem):
    cid = lax.axis_index("c")
    sid = lax.axis_index("s")
    wid = cid * NS + sid
    ebase = wid * EPW

    pltpu.sync_copy(edge_hbm.at[pl.ds(ebase, EPW)], src_v.at[pl.ds(0, EPW)])
    pltpu.sync_copy(edge_hbm.at[pl.ds(E + ebase, EPW)], dst_v.at[pl.ds(0, EPW)])
    pltpu.sync_copy(ab_hbm, ab_v)

    zeros16 = jnp.zeros((16,), jnp.float32)
    for r in range(8):
        for c in range(D // 16):
            rows_v[0, r, pl.ds(c * 16, 16)] = zeros16
    rbase = sid * ROWS_PER_TILE

    def zero_loop(i, carry):
        pltpu.sync_copy(rows_v.at[0, pl.ds(0, 8)],
                        agg_sh.at[pl.ds(rbase + i * 8, 8)])
        return carry

    lax.fori_loop(0, ROWS_PER_TILE // 8, zero_loop, 0)

    def compact(g, off):
        sv = src_v[pl.ds(g * CB, CB)]
        dv = dst_v[pl.ds(g * CB, CB)]
        keep = sv <= dv
        plsc.store_compressed(src_v.at[pl.ds(off, CB)], sv, mask=keep)
        plsc.store_compressed(dst_v.at[pl.ds(off, CB)], dv, mask=keep)
        return off + plsc.all_reduce_population_count(keep)[0]

    off = lax.fori_loop(0, NCHB, compact, 0)
    for k in range(NBUF):
        src_v[pl.ds(off + k * CB, CB)] = jnp.full((CB,), 1, jnp.int32)
        dst_v[pl.ds(off + k * CB, CB)] = jnp.zeros((CB,), jnp.int32)
    nch2 = lax.max((off + CB - 1) // CB, NBUF)
    plsc.subcore_barrier()

    def gather_start(g, b):
        pltpu.async_copy(xm_hbm.at[src_v.at[pl.ds(g * CB, CB)]],
                         rows_v.at[b], gsem.at[b])

    def gather_wait(g, b):
        pltpu.make_async_copy(xm_hbm.at[src_v.at[pl.ds(g * CB, CB)]],
                              rows_v.at[b], gsem.at[b]).wait()

    def scatter_start(b):
        pltpu.async_copy(rows_v.at[b], agg_sh.at[sidx_v.at[b]], ssem.at[b],
                         add=True)

    def scatter_wait(b):
        pltpu.make_async_copy(rows_v.at[b], agg_sh.at[sidx_v.at[b]],
                              ssem.at[b]).wait()

    for b in range(NBUF):
        gather_start(b, b)

    def slot(g, carry):
        b = lax.rem(g, NBUF)
        gather_wait(g, b)
        sv = src_v[pl.ds(g * CB, CB)]
        dv = dst_v[pl.ds(g * CB, CB)]
        av = plsc.load_gather(ab_v, [sv * 2])
        bv = plsc.load_gather(ab_v, [dv * 2 + 1])
        p = 1.0 / (1.0 + jnp.exp(-(av + bv)))
        w = jnp.where(sv <= dv, p, 0.0)
        sidx_v[b, pl.ds(0, CB)] = dv
        for j in range(CB):
            s = w[j]
            for c in range(D // 16):
                rows_v[b, j, pl.ds(c * 16, 16)] = (
                    rows_v[b, j, pl.ds(c * 16, 16)] * s)
        scatter_start(b)
        bp = lax.rem(b + NBUF - 1, NBUF)
        gn = g - 1 + NBUF

        @pl.when(jnp.logical_and(g >= 1, gn < nch2))
        def _():
            scatter_wait(bp)
            gather_start(gn, bp)

        return carry

    lax.fori_loop(0, nch2, slot, 0)
    for b in range(NBUF):
        scatter_wait(b)
    plsc.subcore_barrier()

    pltpu.sync_copy(agg_sh.at[pl.ds(rbase, ROWS_PER_TILE)],
                    out_hbm.at[cid, pl.ds(rbase, ROWS_PER_TILE)])


def _sc_edge(xm, ab, edge_index):
    mesh = plsc.VectorSubcoreMesh(core_axis_name="c", subcore_axis_name="s")
    f = pl.kernel(
        _sc_edge_body,
        out_type=jax.ShapeDtypeStruct((NC, NP, D), jnp.float32),
        mesh=mesh,
        compiler_params=pltpu.CompilerParams(needs_layout_passes=False),
        scratch_types=[
            pltpu.VMEM((EPW + NBUF * CB,), jnp.int32),
            pltpu.VMEM((EPW + NBUF * CB,), jnp.int32),
            pltpu.VMEM((2 * N,), jnp.float32),
            pltpu.VMEM((NBUF, CB, D), jnp.float32),
            pltpu.VMEM((NBUF, CB), jnp.int32),
            pltpu.VMEM_SHARED((NP, D), jnp.float32),
            pltpu.SemaphoreType.DMA((NBUF,)),
            pltpu.SemaphoreType.DMA((NBUF,)),
        ],
    )
    return f(xm, ab, edge_index)


def kernel(x, edge_index, W_rel_src, W_rel_dst, b_rel, W_msg, W_self, W_out,
           b_out):
    da = W_rel_src[:, 1] - W_rel_src[:, 0]
    db = W_rel_dst[:, 1] - W_rel_dst[:, 0]
    wd = jnp.stack([da, db], axis=1)
    c0 = (b_rel[1] - b_rel[0]).reshape(1, 1)
    cvec = jnp.concatenate([c0, jnp.zeros((1, 1), jnp.float32)], axis=1)
    xm, xs, ab = _stage_a(x, W_msg, W_self, wd, cvec)
    agg2 = _sc_edge(xm, ab.reshape(-1), edge_index.reshape(-1))
    return _stage_b(xs, agg2, W_out, b_out.reshape(1, O))


# speedup vs baseline: 25.2100x; 1.1117x over previous
import jax
import jax.numpy as jnp
from jax import lax
from jax.experimental import pallas as pl
from jax.experimental.pallas import tpu as pltpu
from jax.experimental.pallas import tpu_sc as plsc

N = 10000
E = 320000
D = 128
O = 64

NC = 2
NS = 16
NW = NC * NS
EPW = E // NW
CC = 16
CB = 32
NBUF = 4
PAD = NBUF * CB
NCHC = EPW // CC
NP = 10112
ROWS_PER_TILE = NP // NS

BN = 2000


def _stage_a_body(x_ref, wmsg_ref, wself_ref, wd_ref, cvec_ref,
                  xm_ref, xs_ref, ab_ref):
    xb = x_ref[...]
    xm_ref[...] = jnp.dot(xb, wmsg_ref[...], preferred_element_type=jnp.float32)
    xs_ref[...] = jnp.dot(xb, wself_ref[...], preferred_element_type=jnp.float32)
    ab = (jnp.dot(xb, wd_ref[...], preferred_element_type=jnp.float32)
          + cvec_ref[...])
    ab_ref[...] = ab.astype(jnp.bfloat16)


def _stage_a(x, w_msg, w_self, wd, cvec):
    return pl.pallas_call(
        _stage_a_body,
        grid=(N // BN,),
        in_specs=[
            pl.BlockSpec((BN, D), lambda i: (i, 0)),
            pl.BlockSpec((D, D), lambda i: (0, 0)),
            pl.BlockSpec((D, D), lambda i: (0, 0)),
            pl.BlockSpec((D, 2), lambda i: (0, 0)),
            pl.BlockSpec((1, 2), lambda i: (0, 0)),
        ],
        out_specs=[
            pl.BlockSpec((BN, D), lambda i: (i, 0)),
            pl.BlockSpec((BN, D), lambda i: (i, 0)),
            pl.BlockSpec((BN, 2), lambda i: (i, 0)),
        ],
        out_shape=[
            jax.ShapeDtypeStruct((N, D), jnp.float32),
            jax.ShapeDtypeStruct((N, D), jnp.float32),
            jax.ShapeDtypeStruct((N, 2), jnp.bfloat16),
        ],
    )(x, w_msg, w_self, wd, cvec)


def _stage_b_body(xs_ref, agg_ref, wout_ref, bout_ref, out_ref):
    acc = xs_ref[...] + agg_ref[0] + agg_ref[1]
    h = jnp.maximum(acc, 0.0)
    out_ref[...] = (jnp.dot(h, wout_ref[...], preferred_element_type=jnp.float32)
                    + bout_ref[...])


def _stage_b(xs, agg2, w_out, bout2):
    return pl.pallas_call(
        _stage_b_body,
        grid=(N // BN,),
        in_specs=[
            pl.BlockSpec((BN, D), lambda i: (i, 0)),
            pl.BlockSpec((NC, BN, D), lambda i: (0, i, 0)),
            pl.BlockSpec((D, O), lambda i: (0, 0)),
            pl.BlockSpec((1, O), lambda i: (0, 0)),
        ],
        out_specs=pl.BlockSpec((BN, O), lambda i: (i, 0)),
        out_shape=jax.ShapeDtypeStruct((N, O), jnp.float32),
    )(xs, agg2, w_out, bout2)


def _sc_edge_body(xm_hbm, ab_hbm, edge_hbm, out_hbm,
                  src_v, dst_v, ab_v, rows_v, sidx_v, agg_sh, gsem, ssem):
    cid = lax.axis_index("c")
    sid = lax.axis_index("s")
    wid = cid * NS + sid
    ebase = wid * EPW

    pltpu.sync_copy(edge_hbm.at[pl.ds(ebase, EPW)], src_v.at[pl.ds(0, EPW)])
    pltpu.sync_copy(edge_hbm.at[pl.ds(E + ebase, EPW)], dst_v.at[pl.ds(0, EPW)])
    pltpu.sync_copy(ab_hbm, ab_v)

    zeros16 = jnp.zeros((16,), jnp.float32)
    for r in range(8):
        for c in range(D // 16):
            rows_v[0, r, pl.ds(c * 16, 16)] = zeros16
    rbase = sid * ROWS_PER_TILE

    def zero_loop(i, carry):
        pltpu.sync_copy(rows_v.at[0, pl.ds(0, 8)],
                        agg_sh.at[pl.ds(rbase + i * 8, 8)])
        return carry

    lax.fori_loop(0, ROWS_PER_TILE // 8, zero_loop, 0)

    def compact(g, off):
        sv = src_v[pl.ds(g * CC, CC)]
        dv = dst_v[pl.ds(g * CC, CC)]
        keep = sv <= dv
        plsc.store_compressed(src_v.at[pl.ds(off, CC)], sv, mask=keep)
        plsc.store_compressed(dst_v.at[pl.ds(off, CC)], dv, mask=keep)
        return off + plsc.all_reduce_population_count(keep)[0]

    off = lax.fori_loop(0, NCHC, compact, 0)
    for k in range(PAD // CC):
        src_v[pl.ds(off + k * CC, CC)] = jnp.full((CC,), 1, jnp.int32)
        dst_v[pl.ds(off + k * CC, CC)] = jnp.zeros((CC,), jnp.int32)
    nch2 = lax.max((off + CB - 1) // CB, NBUF)
    plsc.subcore_barrier()

    def gather_start(g, b):
        pltpu.async_copy(xm_hbm.at[src_v.at[pl.ds(g * CB, CB)]],
                         rows_v.at[b], gsem.at[b])

    def gather_wait(g, b):
        pltpu.make_async_copy(xm_hbm.at[src_v.at[pl.ds(g * CB, CB)]],
                              rows_v.at[b], gsem.at[b]).wait()

    def scatter_start(b):
        pltpu.async_copy(rows_v.at[b], agg_sh.at[sidx_v.at[b]], ssem.at[b],
                         add=True)

    def scatter_wait(b):
        pltpu.make_async_copy(rows_v.at[b], agg_sh.at[sidx_v.at[b]],
                              ssem.at[b]).wait()

    for b in range(NBUF):
        gather_start(b, b)

    himask = jnp.full((16,), -65536, jnp.int32)

    def slot(g, carry):
        b = lax.rem(g, NBUF)
        gather_wait(g, b)
        for q in range(CB // 16):
            sv = src_v[pl.ds(g * CB + q * 16, 16)]
            dv = dst_v[pl.ds(g * CB + q * 16, 16)]
            g1 = plsc.bitcast(plsc.load_gather(ab_v, [sv]), jnp.int32)
            g2 = plsc.bitcast(plsc.load_gather(ab_v, [dv]), jnp.int32)
            av = plsc.bitcast(lax.shift_left(g1, 16), jnp.float32)
            bv = plsc.bitcast(g2 & himask, jnp.float32)
            p = 1.0 / (1.0 + jnp.exp(-(av + bv)))
            w = jnp.where(sv <= dv, p, 0.0)
            sidx_v[b, pl.ds(q * 16, 16)] = dv
            for j in range(16):
                s = w[j]
                r = q * 16 + j
                for c in range(D // 16):
                    rows_v[b, r, pl.ds(c * 16, 16)] = (
                        rows_v[b, r, pl.ds(c * 16, 16)] * s)
        scatter_start(b)
        bp = lax.rem(b + NBUF - 1, NBUF)
        gn = g - 1 + NBUF

        @pl.when(jnp.logical_and(g >= 1, gn < nch2))
        def _():
            scatter_wait(bp)
            gather_start(gn, bp)

        return carry

    lax.fori_loop(0, nch2, slot, 0)
    for b in range(NBUF):
        scatter_wait(b)
    plsc.subcore_barrier()

    pltpu.sync_copy(agg_sh.at[pl.ds(rbase, ROWS_PER_TILE)],
                    out_hbm.at[cid, pl.ds(rbase, ROWS_PER_TILE)])


def _sc_edge(xm, ab_packed, edge_index):
    mesh = plsc.VectorSubcoreMesh(core_axis_name="c", subcore_axis_name="s")
    f = pl.kernel(
        _sc_edge_body,
        out_type=jax.ShapeDtypeStruct((NC, NP, D), jnp.float32),
        mesh=mesh,
        compiler_params=pltpu.CompilerParams(needs_layout_passes=False),
        scratch_types=[
            pltpu.VMEM((EPW + PAD,), jnp.int32),
            pltpu.VMEM((EPW + PAD,), jnp.int32),
            pltpu.VMEM((N,), jnp.float32),
            pltpu.VMEM((NBUF, CB, D), jnp.float32),
            pltpu.VMEM((NBUF, CB), jnp.int32),
            pltpu.VMEM_SHARED((NP, D), jnp.float32),
            pltpu.SemaphoreType.DMA((NBUF,)),
            pltpu.SemaphoreType.DMA((NBUF,)),
        ],
    )
    return f(xm, ab_packed, edge_index)


def kernel(x, edge_index, W_rel_src, W_rel_dst, b_rel, W_msg, W_self, W_out,
           b_out):
    da = W_rel_src[:, 1] - W_rel_src[:, 0]
    db = W_rel_dst[:, 1] - W_rel_dst[:, 0]
    wd = jnp.stack([da, db], axis=1)
    c0 = (b_rel[1] - b_rel[0]).reshape(1, 1)
    cvec = jnp.concatenate([c0, jnp.zeros((1, 1), jnp.float32)], axis=1)
    xm, xs, ab = _stage_a(x, W_msg, W_self, wd, cvec)
    ab_packed = lax.bitcast_convert_type(ab, jnp.float32)
    agg2 = _sc_edge(xm, ab_packed, edge_index.reshape(-1))
    return _stage_b(xs, agg2, W_out, b_out.reshape(1, O))
